# baseline (device time: 110060 ns/iter reference)
import os

import jax
import jax.numpy as jnp
from jax import lax
from jax.experimental import pallas as pl
from jax.experimental.pallas import tpu as pltpu

N_DEV = 16
SQ = 1024
SKV_LOCAL = 1024
HQ = 8
DH = 128
BLK = 64
CHUNK = SQ // N_DEV
NGRP = 4
GROWS = SQ // NGRP
SCALE = 0.08838834764831843
_SKIP_COMM = bool(os.environ.get("SKIP_COMM"))


def _rem(v):
    return lax.rem(v + 2 * N_DEV, N_DEV)


def kernel(x, Wq, K_ext, V_ext, Wo):
    def body(x_ref, wq_ref, k_ref, v_ref, wo_ref, out_ref,
             acc_ref, l_ref, racc_ref, rl_ref,
             rs_s_sems, rs_r_sems, rsl_s_sems, rsl_r_sems,
             ag_s_sems, ag_r_sems):
        my = lax.axis_index("i")
        left = _rem(my - 1)
        right = _rem(my + 1)

        xb = x_ref[0].astype(jnp.bfloat16)
        wqb = wq_ref[...].astype(jnp.bfloat16)
        q = lax.dot(xb, wqb, preferred_element_type=jnp.float32) * SCALE

        for g in range(NGRP):
            blks = [g + NGRP * b for b in range(NGRP)]
            qg = jnp.concatenate(
                [q[c * BLK:(c + 1) * BLK, :] for c in blks], axis=0)
            for h in range(HQ):
                qgh = qg[:, h * DH:(h + 1) * DH].astype(jnp.bfloat16)
                kgh = jnp.concatenate(
                    [k_ref[0, c * BLK:(c + 1) * BLK, h, :] for c in blks],
                    axis=0).astype(jnp.bfloat16)
                vgh = jnp.concatenate(
                    [v_ref[0, c * BLK:(c + 1) * BLK, h, :] for c in blks],
                    axis=0).astype(jnp.bfloat16)
                s = lax.dot_general(qgh, kgh, (((1,), (1,)), ((), ())),
                                    preferred_element_type=jnp.float32)
                w = jnp.exp(s)
                lgh = jnp.sum(w, axis=1)
                ag = lax.dot(w.astype(jnp.bfloat16), vgh,
                             preferred_element_type=jnp.float32)
                for b in range(NGRP):
                    c = blks[b]
                    acc_ref[c, :, h, :] = ag[b * BLK:(b + 1) * BLK, :]
                    l_ref[c, h, :] = lgh[b * BLK:(b + 1) * BLK]

        barrier = pltpu.get_barrier_semaphore()
        pl.semaphore_signal(barrier, inc=1, device_id=(left,),
                            device_id_type=pl.DeviceIdType.MESH)
        pl.semaphore_signal(barrier, inc=1, device_id=(right,),
                            device_id_type=pl.DeviceIdType.MESH)
        pl.semaphore_wait(barrier, 2)

        for st in range(8 if not _SKIP_COMM else 0):
            cl_s = _rem(my - 8 + st)
            rd_l = pltpu.make_async_remote_copy(
                src_ref=acc_ref.at[cl_s], dst_ref=racc_ref.at[st],
                send_sem=rs_s_sems.at[st], recv_sem=rs_r_sems.at[st],
                device_id=(left,), device_id_type=pl.DeviceIdType.MESH)
            rdl_l = pltpu.make_async_remote_copy(
                src_ref=l_ref.at[cl_s], dst_ref=rl_ref.at[st],
                send_sem=rsl_s_sems.at[st], recv_sem=rsl_r_sems.at[st],
                device_id=(left,), device_id_type=pl.DeviceIdType.MESH)
            rd_l.start()
            rdl_l.start()
            if st < 7:
                cr_s = _rem(my + 7 - st)
                rd_r = pltpu.make_async_remote_copy(
                    src_ref=acc_ref.at[cr_s], dst_ref=racc_ref.at[8 + st],
                    send_sem=rs_s_sems.at[8 + st],
                    recv_sem=rs_r_sems.at[8 + st],
                    device_id=(right,), device_id_type=pl.DeviceIdType.MESH)
                rdl_r = pltpu.make_async_remote_copy(
                    src_ref=l_ref.at[cr_s], dst_ref=rl_ref.at[8 + st],
                    send_sem=rsl_s_sems.at[8 + st],
                    recv_sem=rsl_r_sems.at[8 + st],
                    device_id=(right,), device_id_type=pl.DeviceIdType.MESH)
                rd_r.start()
                rdl_r.start()
            rd_l.wait()
            rdl_l.wait()
            cl = _rem(my - 7 + st)
            acc_ref[cl] = acc_ref[cl] + racc_ref[st]
            l_ref[cl] = l_ref[cl] + rl_ref[st]
            if st < 7:
                rd_r.wait()
                rdl_r.wait()
                cr = _rem(my + 6 - st)
                acc_ref[cr] = acc_ref[cr] + racc_ref[8 + st]
                l_ref[cr] = l_ref[cr] + rl_ref[8 + st]

        accc = acc_ref[my]
        lc = l_ref[my]
        parts = []
        for h in range(HQ):
            parts.append(accc[:, h, :] / lc[h][:, None])
        ctx = jnp.concatenate(parts, axis=1).astype(jnp.bfloat16)
        wob = wo_ref[...].astype(jnp.bfloat16)
        outc = lax.dot(ctx, wob,
                       preferred_element_type=jnp.float32).astype(jnp.bfloat16)
        out_ref[0, pl.ds(my * CHUNK, CHUNK), :] = outc

        for t in range(8 if not _SKIP_COMM else 0):
            g_r = _rem(my - t)
            ag_r = pltpu.make_async_remote_copy(
                src_ref=out_ref.at[0, pl.ds(g_r * CHUNK, CHUNK), :],
                dst_ref=out_ref.at[0, pl.ds(g_r * CHUNK, CHUNK), :],
                send_sem=ag_s_sems.at[t], recv_sem=ag_r_sems.at[t],
                device_id=(right,), device_id_type=pl.DeviceIdType.MESH)
            ag_r.start()
            if t < 7:
                g_l = _rem(my + t)
                ag_l = pltpu.make_async_remote_copy(
                    src_ref=out_ref.at[0, pl.ds(g_l * CHUNK, CHUNK), :],
                    dst_ref=out_ref.at[0, pl.ds(g_l * CHUNK, CHUNK), :],
                    send_sem=ag_s_sems.at[8 + t], recv_sem=ag_r_sems.at[8 + t],
                    device_id=(left,), device_id_type=pl.DeviceIdType.MESH)
                ag_l.start()
            ag_r.wait()
            if t < 7:
                ag_l.wait()

    return pl.pallas_call(
        body,
        out_shape=jax.ShapeDtypeStruct((1, SQ, HQ * DH), jnp.bfloat16),
        in_specs=[pl.BlockSpec(memory_space=pltpu.VMEM)] * 5,
        out_specs=pl.BlockSpec(memory_space=pltpu.VMEM),
        scratch_shapes=[
            pltpu.VMEM((N_DEV, CHUNK, HQ, DH), jnp.float32),
            pltpu.VMEM((N_DEV, HQ, CHUNK), jnp.float32),
            pltpu.VMEM((N_DEV - 1, CHUNK, HQ, DH), jnp.float32),
            pltpu.VMEM((N_DEV - 1, HQ, CHUNK), jnp.float32),
            pltpu.SemaphoreType.DMA((N_DEV - 1,)),
            pltpu.SemaphoreType.DMA((N_DEV - 1,)),
            pltpu.SemaphoreType.DMA((N_DEV - 1,)),
            pltpu.SemaphoreType.DMA((N_DEV - 1,)),
            pltpu.SemaphoreType.DMA((N_DEV - 1,)),
            pltpu.SemaphoreType.DMA((N_DEV - 1,)),
        ],
        compiler_params=pltpu.CompilerParams(
            collective_id=0,
            vmem_limit_bytes=120 * 1024 * 1024,
        ),
    )(x, Wq, K_ext, V_ext, Wo)
